# baseline (speedup 1.0000x reference)
import functools

import jax
import jax.numpy as jnp
from jax import lax
from jax.experimental import pallas as pl
from jax.experimental.pallas import tpu as pltpu
from jax.experimental.pallas import tpu_sc as plsc

N = 10000
E = 320000
D = 128
ABS = 192
C = 64

NC = 2
NS = 16
GRP = 80
ROWS_PER_TILE = 625
OWN_SHIFT = 16

HALF_E = E // NC
BCHUNK = 2000
NBCH = HALF_E // BCHUNK
BCAP = 12288

CHUNK = 384
GRP2 = 128
GPC2 = CHUNK // GRP2
PAIR = 2 * CHUNK

_SC_PARAMS = pltpu.CompilerParams(use_tc_tiling_on_sc=False,
                                  needs_layout_passes=False)
_DN = lax.GatherDimensionNumbers(offset_dims=(), collapsed_slice_dims=(0,),
                                 start_index_map=(0,))


def _bcast(v16, i):
    return lax.gather(v16, jnp.full((16, 1), i, jnp.int32), _DN, (1,),
                      mode=lax.GatherScatterMode.PROMISE_IN_BOUNDS)


@functools.lru_cache(maxsize=None)
def _make_bucket():
    mesh = plsc.VectorSubcoreMesh(core_axis_name="c", subcore_axis_name="s")

    @functools.partial(
        pl.kernel,
        out_type=[
            jax.ShapeDtypeStruct((NC, NS, BCAP), jnp.int32),
            jax.ShapeDtypeStruct((NC, NS, BCAP), jnp.float32),
            jax.ShapeDtypeStruct((NC, NS, 16), jnp.int32),
        ],
        mesh=mesh,
        scratch_types=[
            pltpu.VMEM((BCAP,), jnp.int32),
            pltpu.VMEM((BCAP,), jnp.float32),
            pltpu.VMEM((BCHUNK // GRP, GRP), jnp.int32),
            pltpu.VMEM((BCHUNK // GRP, GRP), jnp.int32),
            pltpu.VMEM((BCHUNK // GRP, GRP), jnp.int32),
            pltpu.VMEM((BCHUNK // GRP, GRP), jnp.int32),
            pltpu.VMEM((BCHUNK,), jnp.float32),
            pltpu.VMEM((BCHUNK,), jnp.float32),
            pltpu.VMEM((16,), jnp.int32),
            pltpu.SemaphoreType.DMA,
            pltpu.SemaphoreType.DMA,
        ],
        compiler_params=_SC_PARAMS,
    )
    def bucket(rows_hbm, cols_hbm, vals_hbm, bp_hbm, bv_hbm, bcnt_hbm,
               bkt_p, bkt_v, rw_a, rw_b, cl_a, cl_b, vl_a, vl_b, cnt_o,
               sa, sb):
        c = lax.axis_index("c")
        s = lax.axis_index("s")
        gbase = c * (HALF_E // GRP)
        iota = lax.iota(jnp.int32, 16)

        def fire_meta(ch, rw, cl, vl, sem):
            goff = gbase + ch * (BCHUNK // GRP)
            pltpu.async_copy(rows_hbm.at[pl.ds(goff, BCHUNK // GRP)], rw,
                             sem)
            pltpu.async_copy(cols_hbm.at[pl.ds(goff, BCHUNK // GRP)], cl,
                             sem)
            pltpu.async_copy(
                vals_hbm.at[pl.ds(c * HALF_E + ch * BCHUNK, BCHUNK)], vl,
                sem)

        def drain_meta(rw, cl, vl, sem):
            pltpu.make_async_copy(rows_hbm.at[pl.ds(0, BCHUNK // GRP)], rw,
                                  sem).wait()
            pltpu.make_async_copy(cols_hbm.at[pl.ds(0, BCHUNK // GRP)], cl,
                                  sem).wait()
            pltpu.make_async_copy(vals_hbm.at[pl.ds(0, BCHUNK)], vl,
                                  sem).wait()

        def consume(rw, cl, vl, cnt_vec):
            def group(g, cnt):
                for k in range(GRP // 16):
                    row = rw[g, pl.ds(k * 16, 16)]
                    col = cl[g, pl.ds(k * 16, 16)]
                    val = vl[pl.ds(g * GRP + k * 16, 16)]
                    owner = (row * 26845) >> 24
                    lrow = row - owner * 625
                    packed = (lrow << OWN_SHIFT) + 2 * col
                    mask = owner == s
                    mi = jnp.where(mask, 1, 0)
                    pref = plsc.cumsum(mi)
                    pos = jnp.minimum(cnt + pref - 1, BCAP - 1)
                    plsc.store_scatter(bkt_p, [pos], packed, mask=mask)
                    plsc.store_scatter(bkt_v, [pos], val, mask=mask)
                    cnt = cnt + _bcast(pref, 15)
                return cnt
            return lax.fori_loop(0, BCHUNK // GRP, group, cnt_vec)

        fire_meta(0, rw_a, cl_a, vl_a, sa)
        fire_meta(1, rw_b, cl_b, vl_b, sb)

        def pair_body(jj, cnt_vec):
            drain_meta(rw_a, cl_a, vl_a, sa)
            cnt_vec = consume(rw_a, cl_a, vl_a, cnt_vec)

            @pl.when(2 * jj + 2 < NBCH)
            def _ra():
                fire_meta(2 * jj + 2, rw_a, cl_a, vl_a, sa)

            drain_meta(rw_b, cl_b, vl_b, sb)
            cnt_vec = consume(rw_b, cl_b, vl_b, cnt_vec)

            @pl.when(2 * jj + 3 < NBCH)
            def _rb():
                fire_meta(2 * jj + 3, rw_b, cl_b, vl_b, sb)

            return cnt_vec

        cnt_vec = lax.fori_loop(0, NBCH // 2, pair_body,
                                jnp.zeros((16,), jnp.int32))

        cnt = jnp.sum(jnp.where(iota == 0, cnt_vec, 0))
        cnt = jnp.minimum(cnt, BCAP - PAIR)
        padded = jnp.maximum(((cnt + PAIR - 1) // PAIR) * PAIR, PAIR)
        null_p = jnp.full((16,), ROWS_PER_TILE << OWN_SHIFT, jnp.int32)
        null_v = jnp.zeros((16,), jnp.float32)

        def pad_block(i, carry):
            off = jnp.minimum(cnt + i * 16 + iota, BCAP - 1)
            plsc.store_scatter(bkt_p, [off], null_p)
            plsc.store_scatter(bkt_v, [off], null_v)
            return carry

        lax.fori_loop(0, (padded - cnt + 15) // 16, pad_block, 0)

        def wr_block(i, carry):
            pltpu.sync_copy(bkt_p.at[pl.ds(i * CHUNK, CHUNK)],
                            bp_hbm.at[c, s, pl.ds(i * CHUNK, CHUNK)])
            pltpu.sync_copy(bkt_v.at[pl.ds(i * CHUNK, CHUNK)],
                            bv_hbm.at[c, s, pl.ds(i * CHUNK, CHUNK)])
            return carry

        lax.fori_loop(0, padded // CHUNK, wr_block, 0)
        cnt_o[pl.ds(0, 16)] = jnp.full((16,), 1, jnp.int32) * padded
        pltpu.sync_copy(cnt_o, bcnt_hbm.at[c, s])

    return bucket


@functools.lru_cache(maxsize=None)
def _make_spmm(f_half: int, idx_base: int, x_rows: int):
    fv = f_half // 16
    mesh = plsc.VectorSubcoreMesh(core_axis_name="c", subcore_axis_name="s")

    @functools.partial(
        pl.kernel,
        out_type=jax.ShapeDtypeStruct((NC, N, f_half), jnp.float32),
        mesh=mesh,
        scratch_types=[
            pltpu.VMEM((BCAP,), jnp.int32),
            pltpu.VMEM((BCAP,), jnp.float32),
            pltpu.VMEM((GPC2, GRP2), jnp.int32),
            pltpu.VMEM((GPC2, GRP2), jnp.int32),
            pltpu.VMEM((CHUNK, f_half), jnp.float32),
            pltpu.VMEM((CHUNK, f_half), jnp.float32),
            pltpu.VMEM((ROWS_PER_TILE + 1, f_half), jnp.float32),
            pltpu.VMEM((16,), jnp.int32),
            pltpu.SemaphoreType.DMA,
            pltpu.SemaphoreType.DMA,
        ],
        compiler_params=_SC_PARAMS,
    )
    def spmm(bp_hbm, bv_hbm, bcnt_hbm, x_hbm, out_hbm,
             meta_p, meta_v, gidx_a, gidx_b, gath_a, gath_b, acc_v, cnt_v,
             sga, sgb):
        c = lax.axis_index("c")
        s = lax.axis_index("s")
        iota = lax.iota(jnp.int32, 16)
        zero = jnp.zeros((16,), jnp.float32)

        def zero_body(i, carry):
            for j in range(fv):
                acc_v[i, pl.ds(j * 16, 16)] = zero
            return carry

        lax.fori_loop(0, ROWS_PER_TILE + 1, zero_body, 0)

        def build_gidx(ch, gidx):
            for g in range(GPC2):
                for k in range(GRP2 // 16):
                    p = meta_p[pl.ds(ch * CHUNK + g * GRP2 + k * 16, 16)]
                    gidx[g, pl.ds(k * 16, 16)] = \
                        (p & 0xFFFF) + (idx_base + c)

        def fire_gathers(gidx, buf, sem):
            for g in range(GPC2):
                pltpu.async_copy(x_hbm.at[gidx.at[g]],
                                 buf.at[pl.ds(g * GRP2, GRP2)], sem)

        def drain_gathers(buf, sem):
            pltpu.make_async_copy(x_hbm.at[pl.ds(0, CHUNK)], buf,
                                  sem).wait()

        def mulacc(ch, buf):
            def block(b, carry):
                e0 = b * 16
                p16 = meta_p[pl.ds(ch * CHUNK + e0, 16)]
                v16 = meta_v[pl.ds(ch * CHUNK + e0, 16)]
                r16 = p16 >> OWN_SHIFT
                for i in range(16):
                    bv = _bcast(v16, i)
                    br = _bcast(r16, i)
                    for j in range(fv):
                        g = buf[e0 + i, pl.ds(j * 16, 16)]
                        plsc.addupdate_scatter(
                            acc_v, [br, j * 16 + iota], g * bv)
                return carry

            lax.fori_loop(0, CHUNK // 16, block, 0)

        for h in range(NC):
            pltpu.sync_copy(bp_hbm.at[h, s], meta_p)
            pltpu.sync_copy(bv_hbm.at[h, s], meta_v)
            pltpu.sync_copy(bcnt_hbm.at[h, s], cnt_v)
            cnt = jnp.sum(jnp.where(iota == 0, cnt_v[pl.ds(0, 16)], 0))
            npairs = cnt // PAIR

            build_gidx(0, gidx_a)
            fire_gathers(gidx_a, gath_a, sga)

            def pair_body(jj, carry):
                ch0 = 2 * jj
                build_gidx(ch0 + 1, gidx_b)
                fire_gathers(gidx_b, gath_b, sgb)
                drain_gathers(gath_a, sga)
                mulacc(ch0, gath_a)

                @pl.when(jj + 1 < npairs)
                def _ra():
                    build_gidx(ch0 + 2, gidx_a)
                    fire_gathers(gidx_a, gath_a, sga)

                drain_gathers(gath_b, sgb)
                mulacc(ch0 + 1, gath_b)
                return carry

            lax.fori_loop(0, npairs, pair_body, 0)

        pltpu.sync_copy(acc_v.at[pl.ds(0, ROWS_PER_TILE)],
                        out_hbm.at[c, pl.ds(s * ROWS_PER_TILE,
                                            ROWS_PER_TILE)])

    return spmm


_BR = 1000


def _tc1(features, Wup, bup):
    def body(x_ref, w_ref, b_ref, r0_ref, xa_ref):
        h = jnp.dot(x_ref[...], w_ref[...],
                    preferred_element_type=jnp.float32,
                    precision=lax.Precision.HIGHEST)
        h = jnp.maximum(h + b_ref[...], 0.0)
        r0_ref[...] = h[:, :64]
        xa_ref[...] = h[:, 64:]

    return pl.pallas_call(
        body,
        grid=(N // _BR,),
        in_specs=[pl.BlockSpec((_BR, D), lambda i: (i, 0)),
                  pl.BlockSpec((D, ABS), lambda i: (0, 0)),
                  pl.BlockSpec((1, ABS), lambda i: (0, 0))],
        out_specs=[pl.BlockSpec((_BR, 64), lambda i: (i, 0)),
                   pl.BlockSpec((_BR, 128), lambda i: (i, 0))],
        out_shape=[jax.ShapeDtypeStruct((N, 64), jnp.float32),
                   jax.ShapeDtypeStruct((N, 128), jnp.float32)],
    )(features, Wup, bup)


def _tc2(r0, sA, uB, Wbot):
    def body(r0_ref, sa_ref, ub_ref, w_ref, g0_ref, xc_ref):
        a1 = jnp.concatenate(
            [r0_ref[...], sa_ref[0], ub_ref[0], ub_ref[1]], axis=1)
        g = jnp.dot(a1, w_ref[...],
                    preferred_element_type=jnp.float32,
                    precision=lax.Precision.HIGHEST)
        g0_ref[...] = g[:, :64]
        xc_ref[...] = g[:, 64:]

    return pl.pallas_call(
        body,
        grid=(N // _BR,),
        in_specs=[pl.BlockSpec((_BR, 64), lambda i: (i, 0)),
                  pl.BlockSpec((NC, _BR, 64), lambda i: (0, i, 0)),
                  pl.BlockSpec((NC, _BR, 32), lambda i: (0, i, 0)),
                  pl.BlockSpec((ABS, ABS), lambda i: (0, 0))],
        out_specs=[pl.BlockSpec((_BR, 64), lambda i: (i, 0)),
                   pl.BlockSpec((_BR, 128), lambda i: (i, 0))],
        out_shape=[jax.ShapeDtypeStruct((N, 64), jnp.float32),
                   jax.ShapeDtypeStruct((N, 128), jnp.float32)],
    )(r0, sA, uB, Wbot)


def _tc3(g0, tC, vD, bb0, bb1, bb2, Wfc, bfc):
    def body(g0_ref, tc_ref, vd_ref, b0_ref, b1_ref, b2_ref, w_ref, bf_ref,
             out_ref):
        a2 = jnp.concatenate(
            [g0_ref[...] + b0_ref[...],
             tc_ref[0] + b1_ref[...],
             jnp.concatenate([vd_ref[0], vd_ref[1]], axis=1) + b2_ref[...]],
            axis=1)
        logits = jnp.dot(a2, w_ref[...],
                         preferred_element_type=jnp.float32,
                         precision=lax.Precision.HIGHEST) + bf_ref[...]
        m = jnp.max(logits, axis=1, keepdims=True)
        ex = jnp.exp(logits - m)
        lse = jnp.log(jnp.sum(ex, axis=1, keepdims=True))
        out_ref[...] = logits - m - lse

    return pl.pallas_call(
        body,
        grid=(N // _BR,),
        in_specs=[pl.BlockSpec((_BR, 64), lambda i: (i, 0)),
                  pl.BlockSpec((NC, _BR, 64), lambda i: (0, i, 0)),
                  pl.BlockSpec((NC, _BR, 32), lambda i: (0, i, 0)),
                  pl.BlockSpec((1, 64), lambda i: (0, 0)),
                  pl.BlockSpec((1, 64), lambda i: (0, 0)),
                  pl.BlockSpec((1, 64), lambda i: (0, 0)),
                  pl.BlockSpec((ABS, C), lambda i: (0, 0)),
                  pl.BlockSpec((1, C), lambda i: (0, 0))],
        out_specs=pl.BlockSpec((_BR, C), lambda i: (i, 0)),
        out_shape=jax.ShapeDtypeStruct((N, C), jnp.float32),
    )(g0, tC, vD, bb0, bb1, bb2, Wfc, bfc)


def kernel(adj_indices, adj_values, features,
           W_up_0, b_up_0, W_up_1, b_up_1, W_up_2, b_up_2,
           W_bot_0, b_bot_0, W_bot_1, b_bot_1, W_bot_2, b_bot_2,
           W_fc, b_fc):
    rows = adj_indices[0].reshape(E // GRP, GRP)
    cols = adj_indices[1].reshape(E // GRP, GRP)

    Wup = jnp.concatenate([W_up_0, W_up_1, W_up_2], axis=1)
    bup = jnp.concatenate([b_up_0, b_up_1, b_up_2], axis=1)
    Wbot = jnp.concatenate([W_bot_0, W_bot_1, W_bot_2], axis=1)

    bp, bv, bcnt = _make_bucket()(rows, cols, adj_values)

    r0, xA = _tc1(features, Wup, bup)

    spmm64 = _make_spmm(64, 0, 2 * N)
    spmm32 = _make_spmm(32, 2 * N, 4 * N)

    sA = spmm64(bp, bv, bcnt, xA.reshape(2 * N, 64))
    uB = spmm32(bp, bv, bcnt, sA.reshape(4 * N, 32))

    g0, xC = _tc2(r0, sA, uB, Wbot)

    tC = spmm64(bp, bv, bcnt, xC.reshape(2 * N, 64))
    vD = spmm32(bp, bv, bcnt, tC.reshape(4 * N, 32))

    return _tc3(g0, tC, vD, b_bot_0, b_bot_1, b_bot_2,
                W_fc, b_fc.reshape(1, C))


# speedup vs baseline: 1.1655x; 1.1655x over previous
import functools

import jax
import jax.numpy as jnp
from jax import lax
from jax.experimental import pallas as pl
from jax.experimental.pallas import tpu as pltpu
from jax.experimental.pallas import tpu_sc as plsc

N = 10000
E = 320000
D = 128
ABS = 192
C = 64

NC = 2
NS = 16
GRP = 80
ROWS_PER_TILE = 625
OWN_SHIFT = 16

HALF_E = E // NC
BCHUNK = 2000
NBCH = HALF_E // BCHUNK
BCAP = 12288

CHUNK = 384
GRP2 = 128
GPC2 = CHUNK // GRP2
PAIR = 2 * CHUNK

_SC_PARAMS = pltpu.CompilerParams(use_tc_tiling_on_sc=False,
                                  needs_layout_passes=False)
_DN = lax.GatherDimensionNumbers(offset_dims=(), collapsed_slice_dims=(0,),
                                 start_index_map=(0,))


def _bcast(v16, i):
    return lax.gather(v16, jnp.full((16, 1), i, jnp.int32), _DN, (1,),
                      mode=lax.GatherScatterMode.PROMISE_IN_BOUNDS)


@functools.lru_cache(maxsize=None)
def _make_bucket():
    mesh = plsc.VectorSubcoreMesh(core_axis_name="c", subcore_axis_name="s")

    @functools.partial(
        pl.kernel,
        out_type=[
            jax.ShapeDtypeStruct((NC, NS, BCAP), jnp.int32),
            jax.ShapeDtypeStruct((NC, NS, BCAP), jnp.float32),
            jax.ShapeDtypeStruct((NC, NS, 16), jnp.int32),
        ],
        mesh=mesh,
        scratch_types=[
            pltpu.VMEM((BCAP,), jnp.int32),
            pltpu.VMEM((BCAP,), jnp.float32),
            pltpu.VMEM((BCHUNK // GRP, GRP), jnp.int32),
            pltpu.VMEM((BCHUNK // GRP, GRP), jnp.int32),
            pltpu.VMEM((BCHUNK // GRP, GRP), jnp.int32),
            pltpu.VMEM((BCHUNK // GRP, GRP), jnp.int32),
            pltpu.VMEM((BCHUNK,), jnp.float32),
            pltpu.VMEM((BCHUNK,), jnp.float32),
            pltpu.VMEM((16,), jnp.int32),
            pltpu.SemaphoreType.DMA,
            pltpu.SemaphoreType.DMA,
        ],
        compiler_params=_SC_PARAMS,
    )
    def bucket(rows_hbm, cols_hbm, vals_hbm, bp_hbm, bv_hbm, bcnt_hbm,
               bkt_p, bkt_v, rw_a, rw_b, cl_a, cl_b, vl_a, vl_b, cnt_o,
               sa, sb):
        c = lax.axis_index("c")
        s = lax.axis_index("s")
        gbase = c * (HALF_E // GRP)
        iota = lax.iota(jnp.int32, 16)

        def fire_meta(ch, rw, cl, vl, sem):
            goff = gbase + ch * (BCHUNK // GRP)
            pltpu.async_copy(rows_hbm.at[pl.ds(goff, BCHUNK // GRP)], rw,
                             sem)
            pltpu.async_copy(cols_hbm.at[pl.ds(goff, BCHUNK // GRP)], cl,
                             sem)
            pltpu.async_copy(
                vals_hbm.at[pl.ds(c * HALF_E + ch * BCHUNK, BCHUNK)], vl,
                sem)

        def drain_meta(rw, cl, vl, sem):
            pltpu.make_async_copy(rows_hbm.at[pl.ds(0, BCHUNK // GRP)], rw,
                                  sem).wait()
            pltpu.make_async_copy(cols_hbm.at[pl.ds(0, BCHUNK // GRP)], cl,
                                  sem).wait()
            pltpu.make_async_copy(vals_hbm.at[pl.ds(0, BCHUNK)], vl,
                                  sem).wait()

        def consume(rw, cl, vl, cnt_vec):
            def group(g, cnt):
                for k in range(GRP // 16):
                    row = rw[g, pl.ds(k * 16, 16)]
                    col = cl[g, pl.ds(k * 16, 16)]
                    val = vl[pl.ds(g * GRP + k * 16, 16)]
                    owner = (row * 26845) >> 24
                    lrow = row - owner * 625
                    packed = (lrow << OWN_SHIFT) + 2 * col
                    mask = owner == s
                    mi = jnp.where(mask, 1, 0)
                    pref = plsc.cumsum(mi)
                    pos = jnp.minimum(cnt + pref - 1, BCAP - 1)
                    plsc.store_scatter(bkt_p, [pos], packed, mask=mask)
                    plsc.store_scatter(bkt_v, [pos], val, mask=mask)
                    cnt = cnt + _bcast(pref, 15)
                return cnt
            return lax.fori_loop(0, BCHUNK // GRP, group, cnt_vec)

        fire_meta(0, rw_a, cl_a, vl_a, sa)
        fire_meta(1, rw_b, cl_b, vl_b, sb)

        def pair_body(jj, cnt_vec):
            drain_meta(rw_a, cl_a, vl_a, sa)
            cnt_vec = consume(rw_a, cl_a, vl_a, cnt_vec)

            @pl.when(2 * jj + 2 < NBCH)
            def _ra():
                fire_meta(2 * jj + 2, rw_a, cl_a, vl_a, sa)

            drain_meta(rw_b, cl_b, vl_b, sb)
            cnt_vec = consume(rw_b, cl_b, vl_b, cnt_vec)

            @pl.when(2 * jj + 3 < NBCH)
            def _rb():
                fire_meta(2 * jj + 3, rw_b, cl_b, vl_b, sb)

            return cnt_vec

        cnt_vec = lax.fori_loop(0, NBCH // 2, pair_body,
                                jnp.zeros((16,), jnp.int32))

        cnt = jnp.sum(jnp.where(iota == 0, cnt_vec, 0))
        cnt = jnp.minimum(cnt, BCAP - PAIR)
        padded = jnp.maximum(((cnt + PAIR - 1) // PAIR) * PAIR, PAIR)
        null_p = jnp.full((16,), ROWS_PER_TILE << OWN_SHIFT, jnp.int32)
        null_v = jnp.zeros((16,), jnp.float32)

        def pad_block(i, carry):
            off = jnp.minimum(cnt + i * 16 + iota, BCAP - 1)
            plsc.store_scatter(bkt_p, [off], null_p)
            plsc.store_scatter(bkt_v, [off], null_v)
            return carry

        lax.fori_loop(0, (padded - cnt + 15) // 16, pad_block, 0)

        def wr_block(i, carry):
            pltpu.sync_copy(bkt_p.at[pl.ds(i * CHUNK, CHUNK)],
                            bp_hbm.at[c, s, pl.ds(i * CHUNK, CHUNK)])
            pltpu.sync_copy(bkt_v.at[pl.ds(i * CHUNK, CHUNK)],
                            bv_hbm.at[c, s, pl.ds(i * CHUNK, CHUNK)])
            return carry

        lax.fori_loop(0, padded // CHUNK, wr_block, 0)
        cnt_o[pl.ds(0, 16)] = jnp.full((16,), 1, jnp.int32) * padded
        pltpu.sync_copy(cnt_o, bcnt_hbm.at[c, s])

    return bucket


@functools.lru_cache(maxsize=None)
def _make_spmm(f_half: int, idx_base: int, x_rows: int):
    fv = f_half // 16
    mesh = plsc.VectorSubcoreMesh(core_axis_name="c", subcore_axis_name="s")

    @functools.partial(
        pl.kernel,
        out_type=jax.ShapeDtypeStruct((NC, N, f_half), jnp.float32),
        mesh=mesh,
        scratch_types=[
            pltpu.VMEM((BCAP,), jnp.int32),
            pltpu.VMEM((BCAP,), jnp.float32),
            pltpu.VMEM((GPC2, GRP2), jnp.int32),
            pltpu.VMEM((GPC2, GRP2), jnp.int32),
            pltpu.VMEM((CHUNK, f_half), jnp.float32),
            pltpu.VMEM((CHUNK, f_half), jnp.float32),
            pltpu.VMEM((ROWS_PER_TILE + 1, f_half), jnp.float32),
            pltpu.VMEM((16,), jnp.int32),
            pltpu.SemaphoreType.DMA,
            pltpu.SemaphoreType.DMA,
        ],
        compiler_params=_SC_PARAMS,
    )
    def spmm(bp_hbm, bv_hbm, bcnt_hbm, x_hbm, out_hbm,
             meta_p, meta_v, gidx_a, gidx_b, gath_a, gath_b, acc_v, cnt_v,
             sga, sgb):
        c = lax.axis_index("c")
        s = lax.axis_index("s")
        iota = lax.iota(jnp.int32, 16)
        zero = jnp.zeros((16,), jnp.float32)

        def zero_body(i, carry):
            for j in range(fv):
                acc_v[i, pl.ds(j * 16, 16)] = zero
            return carry

        lax.fori_loop(0, ROWS_PER_TILE + 1, zero_body, 0)

        def build_gidx(ch, gidx):
            for g in range(GPC2):
                for k in range(GRP2 // 16):
                    p = meta_p[pl.ds(ch * CHUNK + g * GRP2 + k * 16, 16)]
                    gidx[g, pl.ds(k * 16, 16)] = \
                        (p & 0xFFFF) + (idx_base + c)

        def fire_gathers(gidx, buf, sem):
            for g in range(GPC2):
                pltpu.async_copy(x_hbm.at[gidx.at[g]],
                                 buf.at[pl.ds(g * GRP2, GRP2)], sem)

        def drain_gathers(buf, sem):
            pltpu.make_async_copy(x_hbm.at[pl.ds(0, CHUNK)], buf,
                                  sem).wait()

        def mulacc(ch, buf):
            def block(b):
                e0 = b * 16
                p16 = meta_p[pl.ds(ch * CHUNK + e0, 16)]
                v16 = meta_v[pl.ds(ch * CHUNK + e0, 16)]
                r16 = p16 >> OWN_SHIFT
                for i in range(16):
                    bv = _bcast(v16, i)
                    br = _bcast(r16, i)
                    for j in range(fv):
                        g = buf[e0 + i, pl.ds(j * 16, 16)]
                        plsc.addupdate_scatter(
                            acc_v, [br, j * 16 + iota], g * bv)

            plsc.parallel_loop(0, CHUNK // 16, unroll=2)(block)

        for h in range(NC):
            pltpu.sync_copy(bp_hbm.at[h, s], meta_p)
            pltpu.sync_copy(bv_hbm.at[h, s], meta_v)
            pltpu.sync_copy(bcnt_hbm.at[h, s], cnt_v)
            cnt = jnp.sum(jnp.where(iota == 0, cnt_v[pl.ds(0, 16)], 0))
            npairs = cnt // PAIR

            build_gidx(0, gidx_a)
            fire_gathers(gidx_a, gath_a, sga)

            def pair_body(jj, carry):
                ch0 = 2 * jj
                build_gidx(ch0 + 1, gidx_b)
                fire_gathers(gidx_b, gath_b, sgb)
                drain_gathers(gath_a, sga)
                mulacc(ch0, gath_a)

                @pl.when(jj + 1 < npairs)
                def _ra():
                    build_gidx(ch0 + 2, gidx_a)
                    fire_gathers(gidx_a, gath_a, sga)

                drain_gathers(gath_b, sgb)
                mulacc(ch0 + 1, gath_b)
                return carry

            lax.fori_loop(0, npairs, pair_body, 0)

        pltpu.sync_copy(acc_v.at[pl.ds(0, ROWS_PER_TILE)],
                        out_hbm.at[c, pl.ds(s * ROWS_PER_TILE,
                                            ROWS_PER_TILE)])

    return spmm


_BR = 1000


def _tc1(features, Wup, bup):
    def body(x_ref, w_ref, b_ref, r0_ref, xa_ref):
        h = jnp.dot(x_ref[...], w_ref[...],
                    preferred_element_type=jnp.float32,
                    precision=lax.Precision.HIGHEST)
        h = jnp.maximum(h + b_ref[...], 0.0)
        r0_ref[...] = h[:, :64]
        xa_ref[...] = h[:, 64:]

    return pl.pallas_call(
        body,
        grid=(N // _BR,),
        in_specs=[pl.BlockSpec((_BR, D), lambda i: (i, 0)),
                  pl.BlockSpec((D, ABS), lambda i: (0, 0)),
                  pl.BlockSpec((1, ABS), lambda i: (0, 0))],
        out_specs=[pl.BlockSpec((_BR, 64), lambda i: (i, 0)),
                   pl.BlockSpec((_BR, 128), lambda i: (i, 0))],
        out_shape=[jax.ShapeDtypeStruct((N, 64), jnp.float32),
                   jax.ShapeDtypeStruct((N, 128), jnp.float32)],
    )(features, Wup, bup)


def _tc2(r0, sA, uB, Wbot):
    def body(r0_ref, sa_ref, ub_ref, w_ref, g0_ref, xc_ref):
        a1 = jnp.concatenate(
            [r0_ref[...], sa_ref[0], ub_ref[0], ub_ref[1]], axis=1)
        g = jnp.dot(a1, w_ref[...],
                    preferred_element_type=jnp.float32,
                    precision=lax.Precision.HIGHEST)
        g0_ref[...] = g[:, :64]
        xc_ref[...] = g[:, 64:]

    return pl.pallas_call(
        body,
        grid=(N // _BR,),
        in_specs=[pl.BlockSpec((_BR, 64), lambda i: (i, 0)),
                  pl.BlockSpec((NC, _BR, 64), lambda i: (0, i, 0)),
                  pl.BlockSpec((NC, _BR, 32), lambda i: (0, i, 0)),
                  pl.BlockSpec((ABS, ABS), lambda i: (0, 0))],
        out_specs=[pl.BlockSpec((_BR, 64), lambda i: (i, 0)),
                   pl.BlockSpec((_BR, 128), lambda i: (i, 0))],
        out_shape=[jax.ShapeDtypeStruct((N, 64), jnp.float32),
                   jax.ShapeDtypeStruct((N, 128), jnp.float32)],
    )(r0, sA, uB, Wbot)


def _tc3(g0, tC, vD, bb0, bb1, bb2, Wfc, bfc):
    def body(g0_ref, tc_ref, vd_ref, b0_ref, b1_ref, b2_ref, w_ref, bf_ref,
             out_ref):
        a2 = jnp.concatenate(
            [g0_ref[...] + b0_ref[...],
             tc_ref[0] + b1_ref[...],
             jnp.concatenate([vd_ref[0], vd_ref[1]], axis=1) + b2_ref[...]],
            axis=1)
        logits = jnp.dot(a2, w_ref[...],
                         preferred_element_type=jnp.float32,
                         precision=lax.Precision.HIGHEST) + bf_ref[...]
        m = jnp.max(logits, axis=1, keepdims=True)
        ex = jnp.exp(logits - m)
        lse = jnp.log(jnp.sum(ex, axis=1, keepdims=True))
        out_ref[...] = logits - m - lse

    return pl.pallas_call(
        body,
        grid=(N // _BR,),
        in_specs=[pl.BlockSpec((_BR, 64), lambda i: (i, 0)),
                  pl.BlockSpec((NC, _BR, 64), lambda i: (0, i, 0)),
                  pl.BlockSpec((NC, _BR, 32), lambda i: (0, i, 0)),
                  pl.BlockSpec((1, 64), lambda i: (0, 0)),
                  pl.BlockSpec((1, 64), lambda i: (0, 0)),
                  pl.BlockSpec((1, 64), lambda i: (0, 0)),
                  pl.BlockSpec((ABS, C), lambda i: (0, 0)),
                  pl.BlockSpec((1, C), lambda i: (0, 0))],
        out_specs=pl.BlockSpec((_BR, C), lambda i: (i, 0)),
        out_shape=jax.ShapeDtypeStruct((N, C), jnp.float32),
    )(g0, tC, vD, bb0, bb1, bb2, Wfc, bfc)


def kernel(adj_indices, adj_values, features,
           W_up_0, b_up_0, W_up_1, b_up_1, W_up_2, b_up_2,
           W_bot_0, b_bot_0, W_bot_1, b_bot_1, W_bot_2, b_bot_2,
           W_fc, b_fc):
    rows = adj_indices[0].reshape(E // GRP, GRP)
    cols = adj_indices[1].reshape(E // GRP, GRP)

    Wup = jnp.concatenate([W_up_0, W_up_1, W_up_2], axis=1)
    bup = jnp.concatenate([b_up_0, b_up_1, b_up_2], axis=1)
    Wbot = jnp.concatenate([W_bot_0, W_bot_1, W_bot_2], axis=1)

    bp, bv, bcnt = _make_bucket()(rows, cols, adj_values)

    r0, xA = _tc1(features, Wup, bup)

    spmm64 = _make_spmm(64, 0, 2 * N)
    spmm32 = _make_spmm(32, 2 * N, 4 * N)

    sA = spmm64(bp, bv, bcnt, xA.reshape(2 * N, 64))
    uB = spmm32(bp, bv, bcnt, sA.reshape(4 * N, 32))

    g0, xC = _tc2(r0, sA, uB, Wbot)

    tC = spmm64(bp, bv, bcnt, xC.reshape(2 * N, 64))
    vD = spmm32(bp, bv, bcnt, tC.reshape(4 * N, 32))

    return _tc3(g0, tC, vD, b_bot_0, b_bot_1, b_bot_2,
                W_fc, b_fc.reshape(1, C))


# speedup vs baseline: 1.2195x; 1.0463x over previous
import functools

import jax
import jax.numpy as jnp
from jax import lax
from jax.experimental import pallas as pl
from jax.experimental.pallas import tpu as pltpu
from jax.experimental.pallas import tpu_sc as plsc

N = 10000
E = 320000
D = 128
ABS = 192
C = 64

NC = 2
NS = 16
GRP = 80
ROWS_PER_TILE = 625
OWN_SHIFT = 16

HALF_E = E // NC
BCHUNK = 2000
NBCH = HALF_E // BCHUNK
BCAP = 12288

SLAB = 632
CHUNK = 384
GRP2 = 128
GPC2 = CHUNK // GRP2
PAIR = 2 * CHUNK

_SC_PARAMS = pltpu.CompilerParams(use_tc_tiling_on_sc=False,
                                  needs_layout_passes=False)
_DN = lax.GatherDimensionNumbers(offset_dims=(), collapsed_slice_dims=(0,),
                                 start_index_map=(0,))


def _bcast(v16, i):
    return lax.gather(v16, jnp.full((16, 1), i, jnp.int32), _DN, (1,),
                      mode=lax.GatherScatterMode.PROMISE_IN_BOUNDS)


@functools.lru_cache(maxsize=None)
def _make_bucket():
    mesh = plsc.VectorSubcoreMesh(core_axis_name="c", subcore_axis_name="s")

    @functools.partial(
        pl.kernel,
        out_type=[
            jax.ShapeDtypeStruct((NC, NS, BCAP), jnp.int32),
            jax.ShapeDtypeStruct((NC, NS, BCAP), jnp.float32),
            jax.ShapeDtypeStruct((NC, NS, 16), jnp.int32),
        ],
        mesh=mesh,
        scratch_types=[
            pltpu.VMEM((BCAP,), jnp.int32),
            pltpu.VMEM((BCAP,), jnp.float32),
            pltpu.VMEM((BCHUNK // GRP, GRP), jnp.int32),
            pltpu.VMEM((BCHUNK // GRP, GRP), jnp.int32),
            pltpu.VMEM((BCHUNK // GRP, GRP), jnp.int32),
            pltpu.VMEM((BCHUNK // GRP, GRP), jnp.int32),
            pltpu.VMEM((BCHUNK,), jnp.float32),
            pltpu.VMEM((BCHUNK,), jnp.float32),
            pltpu.VMEM((16,), jnp.int32),
            pltpu.SemaphoreType.DMA,
            pltpu.SemaphoreType.DMA,
        ],
        compiler_params=_SC_PARAMS,
    )
    def bucket(rows_hbm, cols_hbm, vals_hbm, bp_hbm, bv_hbm, bcnt_hbm,
               bkt_p, bkt_v, rw_a, rw_b, cl_a, cl_b, vl_a, vl_b, cnt_o,
               sa, sb):
        c = lax.axis_index("c")
        s = lax.axis_index("s")
        gbase = c * (HALF_E // GRP)
        iota = lax.iota(jnp.int32, 16)

        def fire_meta(ch, rw, cl, vl, sem):
            goff = gbase + ch * (BCHUNK // GRP)
            pltpu.async_copy(rows_hbm.at[pl.ds(goff, BCHUNK // GRP)], rw,
                             sem)
            pltpu.async_copy(cols_hbm.at[pl.ds(goff, BCHUNK // GRP)], cl,
                             sem)
            pltpu.async_copy(
                vals_hbm.at[pl.ds(c * HALF_E + ch * BCHUNK, BCHUNK)], vl,
                sem)

        def drain_meta(rw, cl, vl, sem):
            pltpu.make_async_copy(rows_hbm.at[pl.ds(0, BCHUNK // GRP)], rw,
                                  sem).wait()
            pltpu.make_async_copy(cols_hbm.at[pl.ds(0, BCHUNK // GRP)], cl,
                                  sem).wait()
            pltpu.make_async_copy(vals_hbm.at[pl.ds(0, BCHUNK)], vl,
                                  sem).wait()

        def consume(rw, cl, vl, cnt_vec):
            def group(g, cnt):
                for k in range(GRP // 16):
                    row = rw[g, pl.ds(k * 16, 16)]
                    col = cl[g, pl.ds(k * 16, 16)]
                    val = vl[pl.ds(g * GRP + k * 16, 16)]
                    owner = (row * 26845) >> 24
                    lrow = row - owner * 625
                    packed = (lrow << OWN_SHIFT) + 2 * col
                    mask = owner == s
                    mi = jnp.where(mask, 1, 0)
                    pref = plsc.cumsum(mi)
                    pos = jnp.minimum(cnt + pref - 1, BCAP - 1)
                    plsc.store_scatter(bkt_p, [pos], packed, mask=mask)
                    plsc.store_scatter(bkt_v, [pos], val, mask=mask)
                    cnt = cnt + _bcast(pref, 15)
                return cnt
            return lax.fori_loop(0, BCHUNK // GRP, group, cnt_vec)

        fire_meta(0, rw_a, cl_a, vl_a, sa)
        fire_meta(1, rw_b, cl_b, vl_b, sb)

        def pair_body(jj, cnt_vec):
            drain_meta(rw_a, cl_a, vl_a, sa)
            cnt_vec = consume(rw_a, cl_a, vl_a, cnt_vec)

            @pl.when(2 * jj + 2 < NBCH)
            def _ra():
                fire_meta(2 * jj + 2, rw_a, cl_a, vl_a, sa)

            drain_meta(rw_b, cl_b, vl_b, sb)
            cnt_vec = consume(rw_b, cl_b, vl_b, cnt_vec)

            @pl.when(2 * jj + 3 < NBCH)
            def _rb():
                fire_meta(2 * jj + 3, rw_b, cl_b, vl_b, sb)

            return cnt_vec

        cnt_vec = lax.fori_loop(0, NBCH // 2, pair_body,
                                jnp.zeros((16,), jnp.int32))

        cnt = jnp.sum(jnp.where(iota == 0, cnt_vec, 0))
        cnt = jnp.minimum(cnt, BCAP - PAIR)
        padded = jnp.maximum(((cnt + PAIR - 1) // PAIR) * PAIR, PAIR)
        null_p = jnp.full((16,), ROWS_PER_TILE << OWN_SHIFT, jnp.int32)
        null_v = jnp.zeros((16,), jnp.float32)

        def pad_block(i, carry):
            off = jnp.minimum(cnt + i * 16 + iota, BCAP - 1)
            plsc.store_scatter(bkt_p, [off], null_p)
            plsc.store_scatter(bkt_v, [off], null_v)
            return carry

        lax.fori_loop(0, (padded - cnt + 15) // 16, pad_block, 0)

        def wr_block(i, carry):
            pltpu.sync_copy(bkt_p.at[pl.ds(i * CHUNK, CHUNK)],
                            bp_hbm.at[c, s, pl.ds(i * CHUNK, CHUNK)])
            pltpu.sync_copy(bkt_v.at[pl.ds(i * CHUNK, CHUNK)],
                            bv_hbm.at[c, s, pl.ds(i * CHUNK, CHUNK)])
            return carry

        lax.fori_loop(0, padded // CHUNK, wr_block, 0)
        cnt_o[pl.ds(0, 16)] = jnp.full((16,), 1, jnp.int32) * padded
        pltpu.sync_copy(cnt_o, bcnt_hbm.at[c, s])

    return bucket


@functools.lru_cache(maxsize=None)
def _make_spmm(f_half: int, idx_base: int, x_rows: int):
    fv = f_half // 16
    mesh = plsc.VectorSubcoreMesh(core_axis_name="c", subcore_axis_name="s")

    @functools.partial(
        pl.kernel,
        out_type=jax.ShapeDtypeStruct((NC, N, f_half), jnp.float32),
        mesh=mesh,
        scratch_types=[
            pltpu.VMEM((BCAP,), jnp.int32),
            pltpu.VMEM((BCAP,), jnp.float32),
            pltpu.VMEM((GPC2, GRP2), jnp.int32),
            pltpu.VMEM((GPC2, GRP2), jnp.int32),
            pltpu.VMEM((GPC2, GRP2), jnp.int32),
            pltpu.VMEM((GPC2, GRP2), jnp.int32),
            pltpu.VMEM((CHUNK, f_half), jnp.float32),
            pltpu.VMEM((CHUNK, f_half), jnp.float32),
            pltpu.VMEM_SHARED((NS * SLAB, f_half), jnp.float32),
            pltpu.VMEM((16,), jnp.int32),
            pltpu.SemaphoreType.DMA,
            pltpu.SemaphoreType.DMA,
            pltpu.SemaphoreType.DMA,
            pltpu.SemaphoreType.DMA,
        ],
        compiler_params=_SC_PARAMS,
    )
    def spmm(bp_hbm, bv_hbm, bcnt_hbm, x_hbm, out_hbm,
             meta_p, meta_v, gidx_a, gidx_b, ridx_a, ridx_b,
             gath_a, gath_b, acc_v, cnt_v, sga, sgb, ssa, ssb):
        c = lax.axis_index("c")
        s = lax.axis_index("s")
        iota = lax.iota(jnp.int32, 16)
        sbase = s * SLAB
        zero = jnp.zeros((16,), jnp.float32)

        def zero_body(i, carry):
            for j in range(fv):
                gath_a[i, pl.ds(j * 16, 16)] = zero
            return carry

        lax.fori_loop(0, CHUNK, zero_body, 0)
        pltpu.sync_copy(gath_a, acc_v.at[pl.ds(sbase, CHUNK)])
        pltpu.sync_copy(gath_a.at[pl.ds(0, SLAB - CHUNK)],
                        acc_v.at[pl.ds(sbase + CHUNK, SLAB - CHUNK)])

        def build_gidx(ch, gidx, ridx):
            for g in range(GPC2):
                for k in range(GRP2 // 16):
                    p = meta_p[pl.ds(ch * CHUNK + g * GRP2 + k * 16, 16)]
                    gidx[g, pl.ds(k * 16, 16)] = \
                        (p & 0xFFFF) + (idx_base + c)
                    ridx[g, pl.ds(k * 16, 16)] = (p >> OWN_SHIFT) + sbase

        def fire_gathers(gidx, buf, sem):
            for g in range(GPC2):
                pltpu.async_copy(x_hbm.at[gidx.at[g]],
                                 buf.at[pl.ds(g * GRP2, GRP2)], sem)

        def drain_gathers(buf, sem):
            pltpu.make_async_copy(x_hbm.at[pl.ds(0, CHUNK)], buf,
                                  sem).wait()

        def multiply(ch, buf):
            def block(b):
                e0 = b * 16
                v16 = meta_v[pl.ds(ch * CHUNK + e0, 16)]
                for i in range(16):
                    bv = _bcast(v16, i)
                    for j in range(fv):
                        sl = pl.ds(j * 16, 16)
                        buf[e0 + i, sl] = buf[e0 + i, sl] * bv

            plsc.parallel_loop(0, CHUNK // 16, unroll=2)(block)

        def fire_scatters(buf, ridx, sem):
            for g in range(GPC2):
                pltpu.async_copy(buf.at[pl.ds(g * GRP2, GRP2)],
                                 acc_v.at[ridx.at[g]], sem, add=True)

        def drain_scatters(buf, sem):
            pltpu.make_async_copy(x_hbm.at[pl.ds(0, CHUNK)], buf,
                                  sem).wait()

        for h in range(NC):
            pltpu.sync_copy(bp_hbm.at[h, s], meta_p)
            pltpu.sync_copy(bv_hbm.at[h, s], meta_v)
            pltpu.sync_copy(bcnt_hbm.at[h, s], cnt_v)
            cnt = jnp.sum(jnp.where(iota == 0, cnt_v[pl.ds(0, 16)], 0))
            npairs = cnt // PAIR

            build_gidx(0, gidx_a, ridx_a)
            fire_gathers(gidx_a, gath_a, sga)

            def pair_body(jj, carry):
                ch0 = 2 * jj

                @pl.when(jj > 0)
                def _db():
                    drain_scatters(gath_b, ssb)
                build_gidx(ch0 + 1, gidx_b, ridx_b)
                fire_gathers(gidx_b, gath_b, sgb)
                drain_gathers(gath_a, sga)
                multiply(ch0, gath_a)
                fire_scatters(gath_a, ridx_a, ssa)

                @pl.when(jj + 1 < npairs)
                def _ra():
                    drain_scatters(gath_a, ssa)
                    build_gidx(ch0 + 2, gidx_a, ridx_a)
                    fire_gathers(gidx_a, gath_a, sga)

                drain_gathers(gath_b, sgb)
                multiply(ch0 + 1, gath_b)
                fire_scatters(gath_b, ridx_b, ssb)
                return carry

            lax.fori_loop(0, npairs, pair_body, 0)
            drain_scatters(gath_a, ssa)
            drain_scatters(gath_b, ssb)

        pltpu.sync_copy(acc_v.at[pl.ds(sbase, ROWS_PER_TILE)],
                        out_hbm.at[c, pl.ds(s * ROWS_PER_TILE,
                                            ROWS_PER_TILE)])

    return spmm


_BR = 1000


def _tc1(features, Wup, bup):
    def body(x_ref, w_ref, b_ref, r0_ref, xa_ref):
        h = jnp.dot(x_ref[...], w_ref[...],
                    preferred_element_type=jnp.float32,
                    precision=lax.Precision.HIGHEST)
        h = jnp.maximum(h + b_ref[...], 0.0)
        r0_ref[...] = h[:, :64]
        xa_ref[...] = h[:, 64:]

    return pl.pallas_call(
        body,
        grid=(N // _BR,),
        in_specs=[pl.BlockSpec((_BR, D), lambda i: (i, 0)),
                  pl.BlockSpec((D, ABS), lambda i: (0, 0)),
                  pl.BlockSpec((1, ABS), lambda i: (0, 0))],
        out_specs=[pl.BlockSpec((_BR, 64), lambda i: (i, 0)),
                   pl.BlockSpec((_BR, 128), lambda i: (i, 0))],
        out_shape=[jax.ShapeDtypeStruct((N, 64), jnp.float32),
                   jax.ShapeDtypeStruct((N, 128), jnp.float32)],
    )(features, Wup, bup)


def _tc2(r0, sA, uB, Wbot):
    def body(r0_ref, sa_ref, ub_ref, w_ref, g0_ref, xc_ref):
        a1 = jnp.concatenate(
            [r0_ref[...], sa_ref[0], ub_ref[0], ub_ref[1]], axis=1)
        g = jnp.dot(a1, w_ref[...],
                    preferred_element_type=jnp.float32,
                    precision=lax.Precision.HIGHEST)
        g0_ref[...] = g[:, :64]
        xc_ref[...] = g[:, 64:]

    return pl.pallas_call(
        body,
        grid=(N // _BR,),
        in_specs=[pl.BlockSpec((_BR, 64), lambda i: (i, 0)),
                  pl.BlockSpec((NC, _BR, 64), lambda i: (0, i, 0)),
                  pl.BlockSpec((NC, _BR, 32), lambda i: (0, i, 0)),
                  pl.BlockSpec((ABS, ABS), lambda i: (0, 0))],
        out_specs=[pl.BlockSpec((_BR, 64), lambda i: (i, 0)),
                   pl.BlockSpec((_BR, 128), lambda i: (i, 0))],
        out_shape=[jax.ShapeDtypeStruct((N, 64), jnp.float32),
                   jax.ShapeDtypeStruct((N, 128), jnp.float32)],
    )(r0, sA, uB, Wbot)


def _tc3(g0, tC, vD, bb0, bb1, bb2, Wfc, bfc):
    def body(g0_ref, tc_ref, vd_ref, b0_ref, b1_ref, b2_ref, w_ref, bf_ref,
             out_ref):
        a2 = jnp.concatenate(
            [g0_ref[...] + b0_ref[...],
             tc_ref[0] + b1_ref[...],
             jnp.concatenate([vd_ref[0], vd_ref[1]], axis=1) + b2_ref[...]],
            axis=1)
        logits = jnp.dot(a2, w_ref[...],
                         preferred_element_type=jnp.float32,
                         precision=lax.Precision.HIGHEST) + bf_ref[...]
        m = jnp.max(logits, axis=1, keepdims=True)
        ex = jnp.exp(logits - m)
        lse = jnp.log(jnp.sum(ex, axis=1, keepdims=True))
        out_ref[...] = logits - m - lse

    return pl.pallas_call(
        body,
        grid=(N // _BR,),
        in_specs=[pl.BlockSpec((_BR, 64), lambda i: (i, 0)),
                  pl.BlockSpec((NC, _BR, 64), lambda i: (0, i, 0)),
                  pl.BlockSpec((NC, _BR, 32), lambda i: (0, i, 0)),
                  pl.BlockSpec((1, 64), lambda i: (0, 0)),
                  pl.BlockSpec((1, 64), lambda i: (0, 0)),
                  pl.BlockSpec((1, 64), lambda i: (0, 0)),
                  pl.BlockSpec((ABS, C), lambda i: (0, 0)),
                  pl.BlockSpec((1, C), lambda i: (0, 0))],
        out_specs=pl.BlockSpec((_BR, C), lambda i: (i, 0)),
        out_shape=jax.ShapeDtypeStruct((N, C), jnp.float32),
    )(g0, tC, vD, bb0, bb1, bb2, Wfc, bfc)


def kernel(adj_indices, adj_values, features,
           W_up_0, b_up_0, W_up_1, b_up_1, W_up_2, b_up_2,
           W_bot_0, b_bot_0, W_bot_1, b_bot_1, W_bot_2, b_bot_2,
           W_fc, b_fc):
    rows = adj_indices[0].reshape(E // GRP, GRP)
    cols = adj_indices[1].reshape(E // GRP, GRP)

    Wup = jnp.concatenate([W_up_0, W_up_1, W_up_2], axis=1)
    bup = jnp.concatenate([b_up_0, b_up_1, b_up_2], axis=1)
    Wbot = jnp.concatenate([W_bot_0, W_bot_1, W_bot_2], axis=1)

    bp, bv, bcnt = _make_bucket()(rows, cols, adj_values)

    r0, xA = _tc1(features, Wup, bup)

    spmm64 = _make_spmm(64, 0, 2 * N)
    spmm32 = _make_spmm(32, 2 * N, 4 * N)

    sA = spmm64(bp, bv, bcnt, xA.reshape(2 * N, 64))
    uB = spmm32(bp, bv, bcnt, sA.reshape(4 * N, 32))

    g0, xC = _tc2(r0, sA, uB, Wbot)

    tC = spmm64(bp, bv, bcnt, xC.reshape(2 * N, 64))
    vD = spmm32(bp, bv, bcnt, tC.reshape(4 * N, 32))

    return _tc3(g0, tC, vD, b_bot_0, b_bot_1, b_bot_2,
                W_fc, b_fc.reshape(1, C))


# speedup vs baseline: 4.6877x; 3.8439x over previous
import functools

import jax
import jax.numpy as jnp
from jax import lax
from jax.experimental import pallas as pl
from jax.experimental.pallas import tpu as pltpu
from jax.experimental.pallas import tpu_sc as plsc

N = 10000
E = 320000
D = 128
ABS = 192
C = 64

NC = 2
NS = 16
EDGES_PER_TILE = E // NS
GRP = 80
GPC = 5
CHUNK = GRP * GPC
NCHUNKS = EDGES_PER_TILE // CHUNK
NPAIRS = NCHUNKS // 2
GROUPS_PER_TILE = EDGES_PER_TILE // GRP
ROWS_PER_TILE = 624
TAIL_ROWS = N - NS * ROWS_PER_TILE


@functools.lru_cache(maxsize=None)
def _make_spmm(f_half: int, colmul: int, coremul: int):
    fv = f_half // 16
    mesh = plsc.VectorSubcoreMesh(core_axis_name="c", subcore_axis_name="s")

    @functools.partial(
        pl.kernel,
        out_type=jax.ShapeDtypeStruct((NC, N, f_half), jnp.float32),
        mesh=mesh,
        scratch_types=[
            pltpu.VMEM((GROUPS_PER_TILE, GRP), jnp.int32),
            pltpu.VMEM((GPC, GRP), jnp.int32),
            pltpu.VMEM((GPC, GRP), jnp.int32),
            pltpu.VMEM((CHUNK,), jnp.float32),
            pltpu.VMEM((CHUNK,), jnp.float32),
            pltpu.VMEM((CHUNK, f_half), jnp.float32),
            pltpu.VMEM((CHUNK, f_half), jnp.float32),
            pltpu.VMEM_SHARED((N, f_half), jnp.float32),
            pltpu.SemaphoreType.DMA,
            pltpu.SemaphoreType.DMA,
            pltpu.SemaphoreType.DMA,
            pltpu.SemaphoreType.DMA,
        ],
        compiler_params=pltpu.CompilerParams(use_tc_tiling_on_sc=False),
    )
    def spmm(rows_hbm, cols_hbm, vals_hbm, x_hbm, out_hbm,
             gidx_v, ridx_a, ridx_b, vals_a, vals_b, gath_a, gath_b,
             acc_sh, sga, sgb, ssa, ssb):
        c = lax.axis_index("c")
        s = lax.axis_index("s")
        ebase = s * EDGES_PER_TILE
        gbase = s * GROUPS_PER_TILE
        rbase = s * ROWS_PER_TILE

        pltpu.sync_copy(cols_hbm.at[pl.ds(gbase, GROUPS_PER_TILE)], gidx_v)

        def gi_body(i, carry):
            for k in range(GRP // 16):
                sl = pl.ds(k * 16, 16)
                gidx_v[i, sl] = colmul * gidx_v[i, sl] + coremul * c
            return carry

        lax.fori_loop(0, GROUPS_PER_TILE, gi_body, 0)

        zero = jnp.zeros((16,), jnp.float32)

        def zero_body(i, carry):
            for j in range(fv):
                gath_a[i, pl.ds(j * 16, 16)] = zero
            return carry

        lax.fori_loop(0, CHUNK, zero_body, 0)
        pltpu.sync_copy(gath_a, acc_sh.at[pl.ds(rbase, CHUNK)])
        pltpu.sync_copy(gath_a.at[pl.ds(0, ROWS_PER_TILE - CHUNK)],
                        acc_sh.at[pl.ds(rbase + CHUNK,
                                        ROWS_PER_TILE - CHUNK)])

        @pl.when(s == NS - 1)
        def _zero_tail():
            pltpu.sync_copy(gath_a.at[pl.ds(0, TAIL_ROWS)],
                            acc_sh.at[pl.ds(NS * ROWS_PER_TILE, TAIL_ROWS)])

        plsc.subcore_barrier()

        def fire_inbound(ch, buf, rbuf, vbuf, sem):
            for g in range(GPC):
                pltpu.async_copy(x_hbm.at[gidx_v.at[ch * GPC + g]],
                                 buf.at[pl.ds(g * GRP, GRP)], sem)
            pltpu.async_copy(rows_hbm.at[pl.ds(gbase + ch * GPC, GPC)],
                             rbuf, sem)
            pltpu.async_copy(vals_hbm.at[pl.ds(ebase + ch * CHUNK, CHUNK)],
                             vbuf, sem)

        def fire_scatters(buf, rbuf, sem):
            for g in range(GPC):
                pltpu.async_copy(buf.at[pl.ds(g * GRP, GRP)],
                                 acc_sh.at[rbuf.at[g]], sem,
                                 add=True)

        def drain_inbound(buf, rbuf, vbuf, sem):
            pltpu.make_async_copy(x_hbm.at[pl.ds(0, CHUNK)], buf, sem).wait()
            pltpu.make_async_copy(rows_hbm.at[pl.ds(0, GPC)], rbuf,
                                  sem).wait()
            pltpu.make_async_copy(vals_hbm.at[pl.ds(0, CHUNK)], vbuf,
                                  sem).wait()

        def drain_scatters(buf, sem):
            pltpu.make_async_copy(x_hbm.at[pl.ds(0, CHUNK)], buf, sem).wait()

        def multiply(buf, vbuf):
            dn = lax.GatherDimensionNumbers(
                offset_dims=(), collapsed_slice_dims=(0,),
                start_index_map=(0,))

            def mul_block(b, carry2):
                e0 = b * 16
                v16 = vbuf[pl.ds(e0, 16)]
                for i in range(16):
                    bv = lax.gather(
                        v16, jnp.full((16, 1), i, jnp.int32), dn, (1,),
                        mode=lax.GatherScatterMode.PROMISE_IN_BOUNDS)
                    for j in range(fv):
                        sl = pl.ds(j * 16, 16)
                        buf[e0 + i, sl] = buf[e0 + i, sl] * bv
                return carry2

            lax.fori_loop(0, CHUNK // 16, mul_block, 0)

        fire_inbound(0, gath_a, ridx_a, vals_a, sga)
        fire_inbound(1, gath_b, ridx_b, vals_b, sgb)

        def pair_body(jj, carry):
            ch0 = 2 * jj
            ch1 = 2 * jj + 1
            drain_inbound(gath_a, ridx_a, vals_a, sga)
            multiply(gath_a, vals_a)
            fire_scatters(gath_a, ridx_a, ssa)
            drain_inbound(gath_b, ridx_b, vals_b, sgb)
            @pl.when(jj + 1 < NPAIRS)
            def _refill_a():
                drain_scatters(gath_a, ssa)
                fire_inbound(ch0 + 2, gath_a, ridx_a, vals_a, sga)
            multiply(gath_b, vals_b)
            fire_scatters(gath_b, ridx_b, ssb)

            @pl.when(jj + 1 < NPAIRS)
            def _refill_b():
                drain_scatters(gath_b, ssb)
                fire_inbound(ch1 + 2, gath_b, ridx_b, vals_b, sgb)
            return carry

        lax.fori_loop(0, NPAIRS, pair_body, 0)
        drain_scatters(gath_a, ssa)
        drain_scatters(gath_b, ssb)

        plsc.subcore_barrier()
        pltpu.sync_copy(acc_sh.at[pl.ds(rbase, ROWS_PER_TILE)],
                        out_hbm.at[c, pl.ds(rbase, ROWS_PER_TILE)])

        @pl.when(s == NS - 1)
        def _write_tail():
            pltpu.sync_copy(
                acc_sh.at[pl.ds(NS * ROWS_PER_TILE, TAIL_ROWS)],
                out_hbm.at[c, pl.ds(NS * ROWS_PER_TILE, TAIL_ROWS)])

    return spmm


_BR = 1000


def _tc1(features, Wup, bup):
    def body(x_ref, w_ref, b_ref, r0_ref, r1_ref, r2_ref):
        h = jnp.dot(x_ref[...], w_ref[...],
                    preferred_element_type=jnp.float32,
                    precision=lax.Precision.HIGHEST)
        h = jnp.maximum(h + b_ref[...], 0.0)
        r0_ref[...] = h[:, :64]
        r1_ref[...] = h[:, 64:128]
        r2_ref[...] = h[:, 128:]

    return pl.pallas_call(
        body,
        grid=(N // _BR,),
        in_specs=[pl.BlockSpec((_BR, D), lambda i: (i, 0)),
                  pl.BlockSpec((D, ABS), lambda i: (0, 0)),
                  pl.BlockSpec((1, ABS), lambda i: (0, 0))],
        out_specs=[pl.BlockSpec((_BR, 64), lambda i: (i, 0)),
                   pl.BlockSpec((_BR, 64), lambda i: (i, 0)),
                   pl.BlockSpec((_BR, 64), lambda i: (i, 0))],
        out_shape=[jax.ShapeDtypeStruct((N, 64), jnp.float32),
                   jax.ShapeDtypeStruct((N, 64), jnp.float32),
                   jax.ShapeDtypeStruct((N, 64), jnp.float32)],
    )(features, Wup, bup)


def _tc2(r0, s1, u2, Wbot):
    def body(r0_ref, s1_ref, u2_ref, w_ref, g0_ref, g1_ref, g2_ref):
        a1 = jnp.concatenate(
            [r0_ref[...], s1_ref[0], s1_ref[1], u2_ref[0], u2_ref[1]],
            axis=1)
        g = jnp.dot(a1, w_ref[...],
                    preferred_element_type=jnp.float32,
                    precision=lax.Precision.HIGHEST)
        g0_ref[...] = g[:, :64]
        g1_ref[...] = g[:, 64:128]
        g2_ref[...] = g[:, 128:]

    return pl.pallas_call(
        body,
        grid=(N // _BR,),
        in_specs=[pl.BlockSpec((_BR, 64), lambda i: (i, 0)),
                  pl.BlockSpec((NC, _BR, 32), lambda i: (0, i, 0)),
                  pl.BlockSpec((NC, _BR, 32), lambda i: (0, i, 0)),
                  pl.BlockSpec((ABS, ABS), lambda i: (0, 0))],
        out_specs=[pl.BlockSpec((_BR, 64), lambda i: (i, 0)),
                   pl.BlockSpec((_BR, 64), lambda i: (i, 0)),
                   pl.BlockSpec((_BR, 64), lambda i: (i, 0))],
        out_shape=[jax.ShapeDtypeStruct((N, 64), jnp.float32),
                   jax.ShapeDtypeStruct((N, 64), jnp.float32),
                   jax.ShapeDtypeStruct((N, 64), jnp.float32)],
    )(r0, s1, u2, Wbot)


def _tc3(g0, tC, vD, bb0, bb1, bb2, Wfc, bfc):
    def body(g0_ref, tc_ref, vd_ref, b0_ref, b1_ref, b2_ref, w_ref, bf_ref,
             out_ref):
        a2 = jnp.concatenate(
            [g0_ref[...] + b0_ref[...],
             jnp.concatenate([tc_ref[0], tc_ref[1]], axis=1) + b1_ref[...],
             jnp.concatenate([vd_ref[0], vd_ref[1]], axis=1) + b2_ref[...]],
            axis=1)
        logits = jnp.dot(a2, w_ref[...],
                         preferred_element_type=jnp.float32,
                         precision=lax.Precision.HIGHEST) + bf_ref[...]
        m = jnp.max(logits, axis=1, keepdims=True)
        ex = jnp.exp(logits - m)
        lse = jnp.log(jnp.sum(ex, axis=1, keepdims=True))
        out_ref[...] = logits - m - lse

    return pl.pallas_call(
        body,
        grid=(N // _BR,),
        in_specs=[pl.BlockSpec((_BR, 64), lambda i: (i, 0)),
                  pl.BlockSpec((NC, _BR, 32), lambda i: (0, i, 0)),
                  pl.BlockSpec((NC, _BR, 32), lambda i: (0, i, 0)),
                  pl.BlockSpec((1, 64), lambda i: (0, 0)),
                  pl.BlockSpec((1, 64), lambda i: (0, 0)),
                  pl.BlockSpec((1, 64), lambda i: (0, 0)),
                  pl.BlockSpec((ABS, C), lambda i: (0, 0)),
                  pl.BlockSpec((1, C), lambda i: (0, 0))],
        out_specs=pl.BlockSpec((_BR, C), lambda i: (i, 0)),
        out_shape=jax.ShapeDtypeStruct((N, C), jnp.float32),
    )(g0, tC, vD, bb0, bb1, bb2, Wfc, bfc)


def kernel(adj_indices, adj_values, features,
           W_up_0, b_up_0, W_up_1, b_up_1, W_up_2, b_up_2,
           W_bot_0, b_bot_0, W_bot_1, b_bot_1, W_bot_2, b_bot_2,
           W_fc, b_fc):
    rows = adj_indices[0].reshape(E // GRP, GRP)
    cols = adj_indices[1].reshape(E // GRP, GRP)

    Wup = jnp.concatenate([W_up_0, W_up_1, W_up_2], axis=1)
    bup = jnp.concatenate([b_up_0, b_up_1, b_up_2], axis=1)
    Wbot = jnp.concatenate([W_bot_0, W_bot_1, W_bot_2], axis=1)

    r0, r1, r2 = _tc1(features, Wup, bup)

    spmm_i = _make_spmm(32, 2, 1)
    spmm_s = _make_spmm(32, 1, N)

    s1 = spmm_i(rows, cols, adj_values, r1.reshape(2 * N, 32))
    s2 = spmm_i(rows, cols, adj_values, r2.reshape(2 * N, 32))
    u2 = spmm_s(rows, cols, adj_values, s2.reshape(2 * N, 32))

    g0, g1, g2 = _tc2(r0, s1, u2, Wbot)

    t1 = spmm_i(rows, cols, adj_values, g1.reshape(2 * N, 32))
    t2 = spmm_i(rows, cols, adj_values, g2.reshape(2 * N, 32))
    v2 = spmm_s(rows, cols, adj_values, t2.reshape(2 * N, 32))

    return _tc3(g0, t1, v2, b_bot_0, b_bot_1, b_bot_2,
                W_fc, b_fc.reshape(1, C))


# speedup vs baseline: 4.7370x; 1.0105x over previous
import functools

import jax
import jax.numpy as jnp
from jax import lax
from jax.experimental import pallas as pl
from jax.experimental.pallas import tpu as pltpu
from jax.experimental.pallas import tpu_sc as plsc

N = 10000
E = 320000
D = 128
ABS = 192
C = 64

NC = 2
NS = 16
EDGES_PER_TILE = E // NS
GRP = 80
GPC = 10
CHUNK = GRP * GPC
NCHUNKS = EDGES_PER_TILE // CHUNK
NPAIRS = NCHUNKS // 2
GROUPS_PER_TILE = EDGES_PER_TILE // GRP
ROWS_PER_TILE = 624
TAIL_ROWS = N - NS * ROWS_PER_TILE


@functools.lru_cache(maxsize=None)
def _make_spmm(f_half: int, colmul: int, coremul: int):
    fv = f_half // 16
    mesh = plsc.VectorSubcoreMesh(core_axis_name="c", subcore_axis_name="s")

    @functools.partial(
        pl.kernel,
        out_type=jax.ShapeDtypeStruct((NC, N, f_half), jnp.float32),
        mesh=mesh,
        scratch_types=[
            pltpu.VMEM((GROUPS_PER_TILE, GRP), jnp.int32),
            pltpu.VMEM((GPC, GRP), jnp.int32),
            pltpu.VMEM((GPC, GRP), jnp.int32),
            pltpu.VMEM((CHUNK,), jnp.float32),
            pltpu.VMEM((CHUNK,), jnp.float32),
            pltpu.VMEM((CHUNK, f_half), jnp.float32),
            pltpu.VMEM((CHUNK, f_half), jnp.float32),
            pltpu.VMEM_SHARED((N, f_half), jnp.float32),
            pltpu.SemaphoreType.DMA,
            pltpu.SemaphoreType.DMA,
            pltpu.SemaphoreType.DMA,
            pltpu.SemaphoreType.DMA,
        ],
        compiler_params=pltpu.CompilerParams(use_tc_tiling_on_sc=False),
    )
    def spmm(rows_hbm, cols_hbm, vals_hbm, x_hbm, out_hbm,
             gidx_v, ridx_a, ridx_b, vals_a, vals_b, gath_a, gath_b,
             acc_sh, sga, sgb, ssa, ssb):
        c = lax.axis_index("c")
        s = lax.axis_index("s")
        ebase = s * EDGES_PER_TILE
        gbase = s * GROUPS_PER_TILE
        rbase = s * ROWS_PER_TILE

        pltpu.sync_copy(cols_hbm.at[pl.ds(gbase, GROUPS_PER_TILE)], gidx_v)

        def gi_body(i, carry):
            for k in range(GRP // 16):
                sl = pl.ds(k * 16, 16)
                gidx_v[i, sl] = colmul * gidx_v[i, sl] + coremul * c
            return carry

        lax.fori_loop(0, GROUPS_PER_TILE, gi_body, 0)

        zero = jnp.zeros((16,), jnp.float32)

        def zero_body(i, carry):
            for j in range(fv):
                gath_a[i, pl.ds(j * 16, 16)] = zero
            return carry

        lax.fori_loop(0, min(CHUNK, ROWS_PER_TILE), zero_body, 0)
        if CHUNK >= ROWS_PER_TILE:
            pltpu.sync_copy(gath_a.at[pl.ds(0, ROWS_PER_TILE)],
                            acc_sh.at[pl.ds(rbase, ROWS_PER_TILE)])
        else:
            pltpu.sync_copy(gath_a, acc_sh.at[pl.ds(rbase, CHUNK)])
            pltpu.sync_copy(gath_a.at[pl.ds(0, ROWS_PER_TILE - CHUNK)],
                            acc_sh.at[pl.ds(rbase + CHUNK,
                                            ROWS_PER_TILE - CHUNK)])

        @pl.when(s == NS - 1)
        def _zero_tail():
            pltpu.sync_copy(gath_a.at[pl.ds(0, TAIL_ROWS)],
                            acc_sh.at[pl.ds(NS * ROWS_PER_TILE, TAIL_ROWS)])

        plsc.subcore_barrier()

        def fire_inbound(ch, buf, rbuf, vbuf, sem):
            for g in range(GPC):
                pltpu.async_copy(x_hbm.at[gidx_v.at[ch * GPC + g]],
                                 buf.at[pl.ds(g * GRP, GRP)], sem)
            pltpu.async_copy(rows_hbm.at[pl.ds(gbase + ch * GPC, GPC)],
                             rbuf, sem)
            pltpu.async_copy(vals_hbm.at[pl.ds(ebase + ch * CHUNK, CHUNK)],
                             vbuf, sem)

        def fire_scatters(buf, rbuf, sem):
            for g in range(GPC):
                pltpu.async_copy(buf.at[pl.ds(g * GRP, GRP)],
                                 acc_sh.at[rbuf.at[g]], sem,
                                 add=True)

        def drain_inbound(buf, rbuf, vbuf, sem):
            pltpu.make_async_copy(x_hbm.at[pl.ds(0, CHUNK)], buf, sem).wait()
            pltpu.make_async_copy(rows_hbm.at[pl.ds(0, GPC)], rbuf,
                                  sem).wait()
            pltpu.make_async_copy(vals_hbm.at[pl.ds(0, CHUNK)], vbuf,
                                  sem).wait()

        def drain_scatters(buf, sem):
            pltpu.make_async_copy(x_hbm.at[pl.ds(0, CHUNK)], buf, sem).wait()

        def multiply(buf, vbuf):
            dn = lax.GatherDimensionNumbers(
                offset_dims=(), collapsed_slice_dims=(0,),
                start_index_map=(0,))

            def mul_block(b, carry2):
                e0 = b * 16
                v16 = vbuf[pl.ds(e0, 16)]
                for i in range(16):
                    bv = lax.gather(
                        v16, jnp.full((16, 1), i, jnp.int32), dn, (1,),
                        mode=lax.GatherScatterMode.PROMISE_IN_BOUNDS)
                    for j in range(fv):
                        sl = pl.ds(j * 16, 16)
                        buf[e0 + i, sl] = buf[e0 + i, sl] * bv
                return carry2

            lax.fori_loop(0, CHUNK // 16, mul_block, 0)

        fire_inbound(0, gath_a, ridx_a, vals_a, sga)
        fire_inbound(1, gath_b, ridx_b, vals_b, sgb)

        def pair_body(jj, carry):
            ch0 = 2 * jj
            ch1 = 2 * jj + 1
            drain_inbound(gath_a, ridx_a, vals_a, sga)
            multiply(gath_a, vals_a)
            fire_scatters(gath_a, ridx_a, ssa)
            drain_inbound(gath_b, ridx_b, vals_b, sgb)
            @pl.when(jj + 1 < NPAIRS)
            def _refill_a():
                drain_scatters(gath_a, ssa)
                fire_inbound(ch0 + 2, gath_a, ridx_a, vals_a, sga)
            multiply(gath_b, vals_b)
            fire_scatters(gath_b, ridx_b, ssb)

            @pl.when(jj + 1 < NPAIRS)
            def _refill_b():
                drain_scatters(gath_b, ssb)
                fire_inbound(ch1 + 2, gath_b, ridx_b, vals_b, sgb)
            return carry

        lax.fori_loop(0, NPAIRS, pair_body, 0)
        drain_scatters(gath_a, ssa)
        drain_scatters(gath_b, ssb)

        plsc.subcore_barrier()
        pltpu.sync_copy(acc_sh.at[pl.ds(rbase, ROWS_PER_TILE)],
                        out_hbm.at[c, pl.ds(rbase, ROWS_PER_TILE)])

        @pl.when(s == NS - 1)
        def _write_tail():
            pltpu.sync_copy(
                acc_sh.at[pl.ds(NS * ROWS_PER_TILE, TAIL_ROWS)],
                out_hbm.at[c, pl.ds(NS * ROWS_PER_TILE, TAIL_ROWS)])

    return spmm


_BR = 1000


def _tc1(features, Wup, bup):
    def body(x_ref, w_ref, b_ref, r0_ref, r1_ref, r2_ref):
        h = jnp.dot(x_ref[...], w_ref[...],
                    preferred_element_type=jnp.float32,
                    precision=lax.Precision.HIGHEST)
        h = jnp.maximum(h + b_ref[...], 0.0)
        r0_ref[...] = h[:, :64]
        r1_ref[...] = h[:, 64:128]
        r2_ref[...] = h[:, 128:]

    return pl.pallas_call(
        body,
        grid=(N // _BR,),
        in_specs=[pl.BlockSpec((_BR, D), lambda i: (i, 0)),
                  pl.BlockSpec((D, ABS), lambda i: (0, 0)),
                  pl.BlockSpec((1, ABS), lambda i: (0, 0))],
        out_specs=[pl.BlockSpec((_BR, 64), lambda i: (i, 0)),
                   pl.BlockSpec((_BR, 64), lambda i: (i, 0)),
                   pl.BlockSpec((_BR, 64), lambda i: (i, 0))],
        out_shape=[jax.ShapeDtypeStruct((N, 64), jnp.float32),
                   jax.ShapeDtypeStruct((N, 64), jnp.float32),
                   jax.ShapeDtypeStruct((N, 64), jnp.float32)],
    )(features, Wup, bup)


def _tc2(r0, s1, u2, Wbot):
    def body(r0_ref, s1_ref, u2_ref, w_ref, g0_ref, g1_ref, g2_ref):
        a1 = jnp.concatenate(
            [r0_ref[...], s1_ref[0], s1_ref[1], u2_ref[0], u2_ref[1]],
            axis=1)
        g = jnp.dot(a1, w_ref[...],
                    preferred_element_type=jnp.float32,
                    precision=lax.Precision.HIGHEST)
        g0_ref[...] = g[:, :64]
        g1_ref[...] = g[:, 64:128]
        g2_ref[...] = g[:, 128:]

    return pl.pallas_call(
        body,
        grid=(N // _BR,),
        in_specs=[pl.BlockSpec((_BR, 64), lambda i: (i, 0)),
                  pl.BlockSpec((NC, _BR, 32), lambda i: (0, i, 0)),
                  pl.BlockSpec((NC, _BR, 32), lambda i: (0, i, 0)),
                  pl.BlockSpec((ABS, ABS), lambda i: (0, 0))],
        out_specs=[pl.BlockSpec((_BR, 64), lambda i: (i, 0)),
                   pl.BlockSpec((_BR, 64), lambda i: (i, 0)),
                   pl.BlockSpec((_BR, 64), lambda i: (i, 0))],
        out_shape=[jax.ShapeDtypeStruct((N, 64), jnp.float32),
                   jax.ShapeDtypeStruct((N, 64), jnp.float32),
                   jax.ShapeDtypeStruct((N, 64), jnp.float32)],
    )(r0, s1, u2, Wbot)


def _tc3(g0, tC, vD, bb0, bb1, bb2, Wfc, bfc):
    def body(g0_ref, tc_ref, vd_ref, b0_ref, b1_ref, b2_ref, w_ref, bf_ref,
             out_ref):
        a2 = jnp.concatenate(
            [g0_ref[...] + b0_ref[...],
             jnp.concatenate([tc_ref[0], tc_ref[1]], axis=1) + b1_ref[...],
             jnp.concatenate([vd_ref[0], vd_ref[1]], axis=1) + b2_ref[...]],
            axis=1)
        logits = jnp.dot(a2, w_ref[...],
                         preferred_element_type=jnp.float32,
                         precision=lax.Precision.HIGHEST) + bf_ref[...]
        m = jnp.max(logits, axis=1, keepdims=True)
        ex = jnp.exp(logits - m)
        lse = jnp.log(jnp.sum(ex, axis=1, keepdims=True))
        out_ref[...] = logits - m - lse

    return pl.pallas_call(
        body,
        grid=(N // _BR,),
        in_specs=[pl.BlockSpec((_BR, 64), lambda i: (i, 0)),
                  pl.BlockSpec((NC, _BR, 32), lambda i: (0, i, 0)),
                  pl.BlockSpec((NC, _BR, 32), lambda i: (0, i, 0)),
                  pl.BlockSpec((1, 64), lambda i: (0, 0)),
                  pl.BlockSpec((1, 64), lambda i: (0, 0)),
                  pl.BlockSpec((1, 64), lambda i: (0, 0)),
                  pl.BlockSpec((ABS, C), lambda i: (0, 0)),
                  pl.BlockSpec((1, C), lambda i: (0, 0))],
        out_specs=pl.BlockSpec((_BR, C), lambda i: (i, 0)),
        out_shape=jax.ShapeDtypeStruct((N, C), jnp.float32),
    )(g0, tC, vD, bb0, bb1, bb2, Wfc, bfc)


def kernel(adj_indices, adj_values, features,
           W_up_0, b_up_0, W_up_1, b_up_1, W_up_2, b_up_2,
           W_bot_0, b_bot_0, W_bot_1, b_bot_1, W_bot_2, b_bot_2,
           W_fc, b_fc):
    rows = adj_indices[0].reshape(E // GRP, GRP)
    cols = adj_indices[1].reshape(E // GRP, GRP)

    Wup = jnp.concatenate([W_up_0, W_up_1, W_up_2], axis=1)
    bup = jnp.concatenate([b_up_0, b_up_1, b_up_2], axis=1)
    Wbot = jnp.concatenate([W_bot_0, W_bot_1, W_bot_2], axis=1)

    r0, r1, r2 = _tc1(features, Wup, bup)

    spmm_i = _make_spmm(32, 2, 1)
    spmm_s = _make_spmm(32, 1, N)

    s1 = spmm_i(rows, cols, adj_values, r1.reshape(2 * N, 32))
    s2 = spmm_i(rows, cols, adj_values, r2.reshape(2 * N, 32))
    u2 = spmm_s(rows, cols, adj_values, s2.reshape(2 * N, 32))

    g0, g1, g2 = _tc2(r0, s1, u2, Wbot)

    t1 = spmm_i(rows, cols, adj_values, g1.reshape(2 * N, 32))
    t2 = spmm_i(rows, cols, adj_values, g2.reshape(2 * N, 32))
    v2 = spmm_s(rows, cols, adj_values, t2.reshape(2 * N, 32))

    return _tc3(g0, t1, v2, b_bot_0, b_bot_1, b_bot_2,
                W_fc, b_fc.reshape(1, C))
